# baseline (speedup 1.0000x reference)
import functools

import jax
import jax.numpy as jnp
from jax import lax
from jax.experimental import pallas as pl
from jax.experimental.pallas import tpu as pltpu
from jax.experimental.pallas import tpu_sc as plsc

_N_EXPERTS = 16
_TOP_K = 2
_LANES = 128


def _router_body(x_ref, w_ref, o_ref):
    o_ref[...] = lax.dot_general(
        x_ref[...], w_ref[...], (((1,), (1,)), ((), ())),
        preferred_element_type=jnp.float32)


def _router_logits(x_flat, router_w):
    n, d = x_flat.shape
    w_pad = jnp.zeros((_LANES, d), jnp.float32).at[:_N_EXPERTS].set(router_w)
    out = pl.pallas_call(
        _router_body,
        out_shape=jax.ShapeDtypeStruct((n, _LANES), jnp.float32),
    )(x_flat, w_pad)
    return out[:, :_N_EXPERTS]


def _make_sc_gather(n_rows, d, b_per_w):
    info = plsc.get_sparse_core_info()
    mesh = plsc.VectorSubcoreMesh(core_axis_name="c", subcore_axis_name="s")

    chunk = 16
    nchunks = b_per_w // chunk

    @functools.partial(
        pl.kernel,
        out_type=jax.ShapeDtypeStruct((n_rows, d), jnp.float32),
        mesh=mesh,
        scratch_types=[
            pltpu.VMEM((2, chunk), jnp.int32),
            pltpu.VMEM((2, chunk, d), jnp.float32),
            pltpu.SemaphoreType.DMA,
            pltpu.SemaphoreType.DMA,
            pltpu.SemaphoreType.DMA,
            pltpu.SemaphoreType.DMA,
        ],
    )
    def gather(table_hbm, idx_hbm, out_hbm, idx_v, rows_v, g0, g1, o0, o1):
        gsem, osem = (g0, g1), (o0, o1)
        wid = lax.axis_index("s") * info.num_cores + lax.axis_index("c")
        base = wid * b_per_w

        def start_gather(c, b):
            pltpu.sync_copy(
                idx_hbm.at[pl.ds(base + c * chunk, chunk)], idx_v.at[b])
            return pltpu.async_copy(
                table_hbm.at[idx_v.at[b]], rows_v.at[b], gsem[b])

        g = [start_gather(0, 0), None]
        out_cp = [None, None]
        for c in range(nchunks):
            b = c % 2
            nb = 1 - b
            if c + 1 < nchunks:
                if out_cp[nb] is not None:
                    out_cp[nb].wait()
                g[nb] = start_gather(c + 1, nb)
            g[b].wait()
            out_cp[b] = pltpu.async_copy(
                rows_v.at[b], out_hbm.at[pl.ds(base + c * chunk, chunk)],
                osem[b])
        for b in range(2):
            if out_cp[b] is not None:
                out_cp[b].wait()

    return gather


def _mlp_body(s_ref, xs_ref, w1_ref, w2_ref, out_ref):
    p = pl.program_id(0)
    tile = s_ref[0, p]
    rs = s_ref[2, p]
    re = s_ref[3, p]
    anchor = s_ref[4, p]

    @pl.when(anchor == 1)
    def _zero_tile():
        out_ref[...] = jnp.zeros_like(out_ref)

    @pl.when((anchor == 0) & (rs < re))
    def _compute():
        h = lax.dot_general(
            xs_ref[...], w1_ref[0], (((1,), (1,)), ((), ())),
            preferred_element_type=jnp.float32)
        h = jnp.square(jnp.maximum(h, 0.0))
        o = lax.dot_general(
            h, w2_ref[0], (((1,), (1,)), ((), ())),
            preferred_element_type=jnp.float32)
        row = tile * out_ref.shape[0] + lax.broadcasted_iota(
            jnp.int32, (out_ref.shape[0], 1), 0)
        mask = (row >= rs) & (row < re)
        out_ref[...] += jnp.where(mask, o, 0.0)


def kernel(x, router_w, expert_w1, expert_w2):
    bsz, seqlen, hidden = x.shape
    n_tokens = bsz * seqlen
    nk = n_tokens * _TOP_K
    capacity = int(2.0 * nk / _N_EXPERTS)
    x_flat = x.reshape(n_tokens, hidden)

    logits = _router_logits(x_flat, router_w)
    all_weights = jax.nn.softmax(logits, axis=-1)
    topk_w, topk_idx = lax.top_k(all_weights, _TOP_K)
    flat_e = topk_idx.reshape(-1)
    flat_t = jnp.repeat(jnp.arange(n_tokens, dtype=jnp.int32), _TOP_K)
    oh = (flat_e[:, None] == jnp.arange(_N_EXPERTS)[None, :]).astype(jnp.int32)
    csum = jnp.cumsum(oh, axis=0)
    pos = jnp.take_along_axis(csum - oh, flat_e[:, None], axis=1)[:, 0]
    counts = csum[-1]
    offsets = jnp.cumsum(counts) - counts
    rank = offsets[flat_e] + pos
    zi = jnp.zeros((nk,), jnp.int32)
    local_indices = zi.at[rank].set(flat_t)
    weights_flat = jnp.zeros((nk,), jnp.float32).at[rank].set(topk_w.reshape(-1))
    fanout = jnp.full((n_tokens,), float(_TOP_K), jnp.float32)

    gather = _make_sc_gather(nk, hidden, b_per_w=nk // 32)
    xs = gather(x_flat, local_indices)

    tile_rows = capacity
    n_tiles = nk // tile_rows
    n_pieces = 2 * n_tiles + _N_EXPERTS
    seg_start = offsets
    seg_end = offsets + jnp.minimum(counts, capacity)
    ts = jnp.arange(n_tiles, dtype=jnp.int32) * tile_rows
    ov = (seg_start[None, :] < ts[:, None] + tile_rows) & (
        seg_end[None, :] > ts[:, None])
    stride = 32
    t_idx = jnp.arange(n_tiles, dtype=jnp.int32)
    e_idx = jnp.arange(_N_EXPERTS, dtype=jnp.int32)
    key_real = jnp.where(
        ov, t_idx[:, None] * stride + 1 + e_idx[None, :],
        (n_tiles - 1) * stride + stride - 1)
    keys = jnp.sort(jnp.concatenate(
        [t_idx * stride, key_real.reshape(-1)]))[:n_pieces]
    t_arr = keys // stride
    rem = keys % stride
    anchor_arr = (rem == 0).astype(jnp.int32)
    real = (rem >= 1) & (rem <= _N_EXPERTS)
    e_arr = jnp.clip(rem - 1, 0, _N_EXPERTS - 1)
    rs_arr = jnp.where(real, jnp.maximum(seg_start[e_arr], t_arr * tile_rows), 0)
    re_arr = jnp.where(
        real, jnp.minimum(seg_end[e_arr], t_arr * tile_rows + tile_rows), 0)
    def _fill(carry, re_pair):
        is_real, e_val = re_pair
        carry = jnp.where(is_real, e_val, carry)
        return carry, carry
    _, e_rev = lax.scan(
        _fill, jnp.int32(_N_EXPERTS - 1),
        (jnp.flip(real), jnp.flip(e_arr)))
    e_arr = jnp.flip(e_rev)
    desc = jnp.stack([t_arr, e_arr, rs_arr, re_arr, anchor_arr]).astype(jnp.int32)

    f_dim = expert_w1.shape[1]
    out_sorted = pl.pallas_call(
        _mlp_body,
        grid_spec=pltpu.PrefetchScalarGridSpec(
            num_scalar_prefetch=1,
            grid=(n_pieces,),
            in_specs=[
                pl.BlockSpec((tile_rows, hidden), lambda p, s: (s[0, p], 0)),
                pl.BlockSpec((1, f_dim, hidden), lambda p, s: (s[1, p], 0, 0)),
                pl.BlockSpec((1, hidden, f_dim), lambda p, s: (s[1, p], 0, 0)),
            ],
            out_specs=pl.BlockSpec((tile_rows, hidden), lambda p, s: (s[0, p], 0)),
        ),
        out_shape=jax.ShapeDtypeStruct((nk, hidden), jnp.float32),
    )(desc, xs, expert_w1, expert_w2)

    return (out_sorted, local_indices, weights_flat, fanout)


# speedup vs baseline: 1.3430x; 1.3430x over previous
import functools

import jax
import jax.numpy as jnp
from jax import lax
from jax.experimental import pallas as pl
from jax.experimental.pallas import tpu as pltpu
from jax.experimental.pallas import tpu_sc as plsc

_N_EXPERTS = 16
_TOP_K = 2
_LANES = 128


def _router_body(x_ref, w_ref, o_ref):
    o_ref[...] = lax.dot_general(
        x_ref[...], w_ref[...], (((1,), (1,)), ((), ())),
        preferred_element_type=jnp.float32)


def _router_logits(x_flat, router_w):
    n, d = x_flat.shape
    w_pad = jnp.zeros((_LANES, d), jnp.float32).at[:_N_EXPERTS].set(router_w)
    out = pl.pallas_call(
        _router_body,
        out_shape=jax.ShapeDtypeStruct((n, _LANES), jnp.float32),
    )(x_flat, w_pad)
    return out[:, :_N_EXPERTS]


def _make_sc_gather(n_rows, d, b_per_w):
    info = plsc.get_sparse_core_info()
    mesh = plsc.VectorSubcoreMesh(core_axis_name="c", subcore_axis_name="s")

    chunk = 16
    nchunks = b_per_w // chunk

    @functools.partial(
        pl.kernel,
        out_type=jax.ShapeDtypeStruct((n_rows, d), jnp.float32),
        mesh=mesh,
        scratch_types=[
            pltpu.VMEM((2, chunk), jnp.int32),
            pltpu.VMEM((2, chunk, d), jnp.float32),
            pltpu.SemaphoreType.DMA,
            pltpu.SemaphoreType.DMA,
            pltpu.SemaphoreType.DMA,
            pltpu.SemaphoreType.DMA,
        ],
    )
    def gather(table_hbm, idx_hbm, out_hbm, idx_v, rows_v, g0, g1, o0, o1):
        gsem, osem = (g0, g1), (o0, o1)
        wid = lax.axis_index("s") * info.num_cores + lax.axis_index("c")
        base = wid * b_per_w

        def start_gather(c, b):
            pltpu.sync_copy(
                idx_hbm.at[pl.ds(base + c * chunk, chunk)], idx_v.at[b])
            return pltpu.async_copy(
                table_hbm.at[idx_v.at[b]], rows_v.at[b], gsem[b])

        g = [start_gather(0, 0), None]
        out_cp = [None, None]
        for c in range(nchunks):
            b = c % 2
            nb = 1 - b
            if c + 1 < nchunks:
                if out_cp[nb] is not None:
                    out_cp[nb].wait()
                g[nb] = start_gather(c + 1, nb)
            g[b].wait()
            out_cp[b] = pltpu.async_copy(
                rows_v.at[b], out_hbm.at[pl.ds(base + c * chunk, chunk)],
                osem[b])
        for b in range(2):
            if out_cp[b] is not None:
                out_cp[b].wait()

    return gather


def _mlp_body(s_ref, xs_ref, w1_ref, w2_ref, out_ref):
    p = pl.program_id(0)
    tile = s_ref[0, p]
    rs = s_ref[2, p]
    re = s_ref[3, p]
    anchor = s_ref[4, p]

    @pl.when(anchor == 1)
    def _zero_tile():
        out_ref[...] = jnp.zeros_like(out_ref)

    @pl.when((anchor == 0) & (rs < re))
    def _compute():
        h = lax.dot_general(
            xs_ref[...], w1_ref[0], (((1,), (1,)), ((), ())),
            preferred_element_type=jnp.float32)
        h = jnp.square(jnp.maximum(h, 0.0))
        o = lax.dot_general(
            h, w2_ref[0], (((1,), (1,)), ((), ())),
            preferred_element_type=jnp.float32)
        row = tile * out_ref.shape[0] + lax.broadcasted_iota(
            jnp.int32, (out_ref.shape[0], 1), 0)
        mask = (row >= rs) & (row < re)
        out_ref[...] += jnp.where(mask, o, 0.0)


def kernel(x, router_w, expert_w1, expert_w2):
    bsz, seqlen, hidden = x.shape
    n_tokens = bsz * seqlen
    nk = n_tokens * _TOP_K
    capacity = int(2.0 * nk / _N_EXPERTS)
    x_flat = x.reshape(n_tokens, hidden)

    logits = _router_logits(x_flat, router_w)
    all_weights = jax.nn.softmax(logits, axis=-1)
    topk_w, topk_idx = lax.top_k(all_weights, _TOP_K)
    flat_e = topk_idx.reshape(-1)
    flat_t = jnp.repeat(jnp.arange(n_tokens, dtype=jnp.int32), _TOP_K)
    oh = (flat_e[:, None] == jnp.arange(_N_EXPERTS)[None, :]).astype(jnp.int32)
    csum = jnp.cumsum(oh, axis=0)
    pos = jnp.take_along_axis(csum - oh, flat_e[:, None], axis=1)[:, 0]
    counts = csum[-1]
    offsets = jnp.cumsum(counts) - counts
    rank = offsets[flat_e] + pos
    zi = jnp.zeros((nk,), jnp.int32)
    local_indices = zi.at[rank].set(flat_t)
    weights_flat = jnp.zeros((nk,), jnp.float32).at[rank].set(topk_w.reshape(-1))
    fanout = jnp.full((n_tokens,), float(_TOP_K), jnp.float32)

    gather = _make_sc_gather(nk, hidden, b_per_w=nk // 32)
    xs = gather(x_flat, local_indices)

    tile_rows = capacity
    n_tiles = nk // tile_rows
    n_pieces = 2 * n_tiles + _N_EXPERTS
    seg_start = offsets
    seg_end = offsets + jnp.minimum(counts, capacity)
    ts = jnp.arange(n_tiles, dtype=jnp.int32) * tile_rows
    ov = (seg_start[None, :] < ts[:, None] + tile_rows) & (
        seg_end[None, :] > ts[:, None])
    stride = 32
    t_idx = jnp.arange(n_tiles, dtype=jnp.int32)
    e_idx = jnp.arange(_N_EXPERTS, dtype=jnp.int32)
    key_real = jnp.where(
        ov, t_idx[:, None] * stride + 1 + e_idx[None, :],
        (n_tiles - 1) * stride + stride - 1)
    keys = jnp.sort(jnp.concatenate(
        [t_idx * stride, key_real.reshape(-1)]))[:n_pieces]
    t_arr = keys // stride
    rem = keys % stride
    anchor_arr = (rem == 0).astype(jnp.int32)
    real = (rem >= 1) & (rem <= _N_EXPERTS)
    e_arr = jnp.clip(rem - 1, 0, _N_EXPERTS - 1)
    rs_arr = jnp.where(real, jnp.maximum(seg_start[e_arr], t_arr * tile_rows), 0)
    re_arr = jnp.where(
        real, jnp.minimum(seg_end[e_arr], t_arr * tile_rows + tile_rows), 0)
    rev_real = jnp.flip(real)
    rev_e = jnp.flip(e_arr)
    pidx = jnp.arange(n_pieces, dtype=jnp.int32)
    last_real = lax.cummax(jnp.where(rev_real, pidx, -1))
    e_fill_rev = jnp.where(
        last_real >= 0, rev_e[jnp.maximum(last_real, 0)], _N_EXPERTS - 1)
    e_arr = jnp.flip(e_fill_rev)
    desc = jnp.stack([t_arr, e_arr, rs_arr, re_arr, anchor_arr]).astype(jnp.int32)

    f_dim = expert_w1.shape[1]
    out_sorted = pl.pallas_call(
        _mlp_body,
        grid_spec=pltpu.PrefetchScalarGridSpec(
            num_scalar_prefetch=1,
            grid=(n_pieces,),
            in_specs=[
                pl.BlockSpec((tile_rows, hidden), lambda p, s: (s[0, p], 0)),
                pl.BlockSpec((1, f_dim, hidden), lambda p, s: (s[1, p], 0, 0)),
                pl.BlockSpec((1, hidden, f_dim), lambda p, s: (s[1, p], 0, 0)),
            ],
            out_specs=pl.BlockSpec((tile_rows, hidden), lambda p, s: (s[0, p], 0)),
        ),
        out_shape=jax.ShapeDtypeStruct((nk, hidden), jnp.float32),
    )(desc, xs, expert_w1, expert_w2)

    return (out_sorted, local_indices, weights_flat, fanout)
